# baseline (speedup 1.0000x reference)
import functools

import jax
import jax.numpy as jnp
from jax import lax
from jax.experimental import pallas as pl
from jax.experimental.pallas import tpu as pltpu
from jax.experimental.pallas import tpu_sc as plsc

B, T, C = 32, 1024, 768
CHUNK = 32
NCH = T // CHUNK
L = 16
NSUB = 16


def _loc_ed_body(img_hbm, idx_hbm, out_hbm, idx_v, inv_v, ichunk,
                 tbufs, sbuf, gsems, xsems, wsems):
    cid = lax.axis_index("c")
    sid = lax.axis_index("s")
    b = sid * 2 + cid

    pltpu.sync_copy(idx_hbm, idx_v)
    lanes = lax.broadcasted_iota(jnp.int32, (L,), 0)
    for k in range(T // L):
        v = idx_v[pl.ds(k * L, L)]
        plsc.store_scatter(inv_v, [v], lanes + k * L)

    wr = [None, None]
    for j in range(NCH):
        k = j % 2
        for k2 in range(CHUNK // L):
            ichunk[pl.ds(k2 * L, L)] = inv_v[pl.ds(j * CHUNK + k2 * L, L)]
        pltpu.async_copy(img_hbm.at[b].at[ichunk], tbufs[k], gsems[k]).wait()
        if wr[k] is not None:
            wr[k].wait()
        pltpu.async_copy(tbufs[k], sbuf.at[sid, k], xsems[k]).wait()
        wr[k] = pltpu.async_copy(
            sbuf.at[sid, k], out_hbm.at[b].at[pl.ds(j * CHUNK, CHUNK)],
            wsems[k])
    wr[0].wait()
    wr[1].wait()


@functools.partial(
    pl.kernel,
    out_type=jax.ShapeDtypeStruct((B, T, C), jnp.float32),
    mesh=plsc.VectorSubcoreMesh(core_axis_name="c", subcore_axis_name="s"),
    compiler_params=pltpu.CompilerParams(needs_layout_passes=False),
    scratch_types=[
        pltpu.VMEM((T,), jnp.int32),
        pltpu.VMEM((T,), jnp.int32),
        pltpu.VMEM((CHUNK,), jnp.int32),
        [pltpu.VMEM((CHUNK, C), jnp.float32) for _ in range(2)],
        pltpu.VMEM_SHARED((NSUB, 2, CHUNK, C), jnp.float32),
        [pltpu.SemaphoreType.DMA for _ in range(2)],
        [pltpu.SemaphoreType.DMA for _ in range(2)],
        [pltpu.SemaphoreType.DMA for _ in range(2)],
    ],
)
def _loc_ed_sc(img_hbm, idx_hbm, out_hbm, idx_v, inv_v, ichunk,
               tbufs, sbuf, gsems, xsems, wsems):
    _loc_ed_body(img_hbm, idx_hbm, out_hbm, idx_v, inv_v, ichunk,
                 tbufs, sbuf, gsems, xsems, wsems)


def kernel(img, index_flat_inv):
    idx32 = index_flat_inv.astype(jnp.int32)
    return _loc_ed_sc(img, idx32)
